# baseline (speedup 1.0000x reference)
import dataclasses
import functools

import jax
import jax.numpy as jnp
from jax import lax
from jax.experimental import pallas as pl
from jax.experimental.pallas import tpu as pltpu
from jax.experimental.pallas import tpu_sc as plsc

B, L, D, MAX_LEN = 4, 2048, 1024, 2048
N = B * L
NUM_WORKERS = 32
ROWS_PER_WORKER = N // NUM_WORKERS
CHUNK = 32
LANES = 16

_CP = pltpu.CompilerParams()
if "needs_layout_passes" in pltpu.CompilerParams.__dataclass_fields__:
    _CP = dataclasses.replace(_CP, needs_layout_passes=False)


@functools.partial(
    pl.kernel,
    out_type=jax.ShapeDtypeStruct((N * D,), jnp.float32),
    mesh=plsc.VectorSubcoreMesh(core_axis_name="c", subcore_axis_name="s"),
    compiler_params=_CP,
    scratch_types=[
        pltpu.VMEM((CHUNK,), jnp.int32),
        pltpu.VMEM((CHUNK,), jnp.float32),
        pltpu.VMEM((CHUNK * D,), jnp.float32),
        pltpu.VMEM((CHUNK, D), jnp.float32),
        pltpu.SemaphoreType.DMA,
        pltpu.SemaphoreType.DMA,
    ],
)
def _sc_masked_pe(x_hbm, mask_hbm, pos_hbm, pe_hbm, out_hbm,
                  idx_v, msk_v, x_v, pe_v, sem_a, sem_b):
    wid = lax.axis_index("s") * 2 + lax.axis_index("c")
    base = wid * ROWS_PER_WORKER

    @pl.loop(0, ROWS_PER_WORKER, step=CHUNK)
    def _chunk(off):
        rb = base + off
        pltpu.sync_copy(pos_hbm.at[pl.ds(rb, CHUNK)], idx_v)
        pltpu.sync_copy(mask_hbm.at[pl.ds(rb, CHUNK)], msk_v)
        gat = pltpu.async_copy(pe_hbm.at[idx_v], pe_v, sem_a)
        lin = pltpu.async_copy(x_hbm.at[pl.ds(rb * D, CHUNK * D)], x_v, sem_b)
        gat.wait()
        lin.wait()

        @pl.loop(0, CHUNK)
        def _row(r):
            m = plsc.load_gather(msk_v, [jnp.full((LANES,), r, jnp.int32)])

            @pl.loop(0, D, step=LANES)
            def _col(j):
                xs = pl.ds(r * D + j, LANES)
                x_v[xs] = x_v[xs] + m * pe_v[r, pl.ds(j, LANES)]

        pltpu.sync_copy(x_v, out_hbm.at[pl.ds(rb * D, CHUNK * D)])


@jax.jit
def kernel(x, source_mask, positions, positional_encoding):
    x2 = x.reshape(N * D)
    mask = source_mask.reshape(N).astype(jnp.float32)
    pos = positions.reshape(N).astype(jnp.int32)
    out = _sc_masked_pe(x2, mask, pos, positional_encoding)
    return out.reshape(B, L, D)
